# Initial kernel scaffold; baseline (speedup 1.0000x reference)
import jax
import jax.numpy as jnp
from jax.experimental import pallas as pl


def kernel(query_indices, seq_indices, table):
    raise NotImplementedError("write your pallas kernel here")


# speedup vs baseline: 2.1917x; 2.1917x over previous
import functools

import jax
import jax.numpy as jnp
from jax import lax
from jax.experimental import pallas as pl
from jax.experimental.pallas import tpu as pltpu
from jax.experimental.pallas import tpu_sc as plsc

CH = 128
G = 8


@functools.lru_cache(maxsize=None)
def _build(nchunk: int, d: int, v: int, nw: int, nc: int):
    n_groups, tail = divmod(nchunk, G)
    mesh = plsc.VectorSubcoreMesh(core_axis_name="c", subcore_axis_name="s")

    @functools.partial(
        pl.kernel,
        mesh=mesh,
        compiler_params=pltpu.CompilerParams(use_tc_tiling_on_sc=False),
        out_type=jax.ShapeDtypeStruct((nw, nchunk, CH, d), jnp.float32),
        scratch_types=[
            pltpu.VMEM((nchunk, CH), jnp.int32),
            pltpu.VMEM((G, CH, d), jnp.float32),
            pltpu.SemaphoreType.DMA,
        ],
    )
    def gather_k(idx_hbm, table_hbm, out_hbm, idx_v, rows_v, sem):
        wid = lax.axis_index("s") * nc + lax.axis_index("c")
        pltpu.sync_copy(idx_hbm.at[wid], idx_v)

        def do_group(g, nch):
            copies = [
                pltpu.async_copy(
                    table_hbm.at[idx_v.at[g * G + j]], rows_v.at[j], sem
                )
                for j in range(nch)
            ]
            for c in copies:
                c.wait()
            pltpu.sync_copy(
                rows_v.at[pl.ds(0, nch)], out_hbm.at[wid, pl.ds(g * G, nch)]
            )

        def body(g, carry):
            do_group(g, G)
            return carry

        lax.fori_loop(0, n_groups, body, 0)
        if tail:
            do_group(n_groups, tail)

    return gather_k


def kernel(query_indices, seq_indices, table):
    b = query_indices.shape[0]
    v, d = table.shape
    idx = jnp.concatenate(
        [query_indices.astype(jnp.int32), seq_indices.astype(jnp.int32)], axis=1
    )
    total = idx.size
    info = plsc.get_sparse_core_info()
    nc, ns = info.num_cores, info.num_subcores
    nw = nc * ns
    assert total % (nw * CH) == 0
    nchunk = total // (nw * CH)
    idx3 = idx.reshape(nw, nchunk, CH)
    out = _build(nchunk, d, v, nw, nc)(idx3, table)
    return out.reshape(b, -1)


# speedup vs baseline: 2.2039x; 1.0056x over previous
import functools

import jax
import jax.numpy as jnp
from jax import lax
from jax.experimental import pallas as pl
from jax.experimental.pallas import tpu as pltpu
from jax.experimental.pallas import tpu_sc as plsc

CH = 128
NBUF = 4


@functools.lru_cache(maxsize=None)
def _build(nchunk: int, d: int, v: int, nw: int, nc: int):
    n_main = (nchunk // NBUF) * NBUF
    tail = nchunk - n_main
    mesh = plsc.VectorSubcoreMesh(core_axis_name="c", subcore_axis_name="s")

    @functools.partial(
        pl.kernel,
        mesh=mesh,
        compiler_params=pltpu.CompilerParams(use_tc_tiling_on_sc=False),
        out_type=jax.ShapeDtypeStruct((nw, nchunk, CH, d), jnp.float32),
        scratch_types=[
            pltpu.VMEM((nchunk, CH), jnp.int32),
            pltpu.VMEM((NBUF, CH, d), jnp.float32),
            pltpu.SemaphoreType.DMA((NBUF,)),
            pltpu.SemaphoreType.DMA((NBUF,)),
        ],
    )
    def gather_k(idx_hbm, table_hbm, out_hbm, idx_v, rows_v, sem_g, sem_s):
        wid = lax.axis_index("s") * nc + lax.axis_index("c")
        pltpu.sync_copy(idx_hbm.at[wid], idx_v)

        def fire_gather(ch, b):
            return pltpu.async_copy(
                table_hbm.at[idx_v.at[ch]], rows_v.at[b], sem_g.at[b]
            )

        def fire_store(ch, b):
            return pltpu.async_copy(
                rows_v.at[b], out_hbm.at[wid, ch], sem_s.at[b]
            )

        def wait_store(b):
            pltpu.make_async_copy(
                rows_v.at[b], out_hbm.at[wid, 0], sem_s.at[b]
            ).wait()

        def step(base, nch, first):
            gathers = []
            for b in range(nch):
                if not first:
                    wait_store(b)
                gathers.append(fire_gather(base + b, b))
            for b in range(nch):
                gathers[b].wait()
                fire_store(base + b, b)

        step(0, NBUF, True)

        def body(i, carry):
            step(i * NBUF, NBUF, False)
            return carry

        lax.fori_loop(1, n_main // NBUF, body, 0)
        if tail:
            step(n_main, tail, False)
        for b in range(NBUF):
            wait_store(b)

    return gather_k


def kernel(query_indices, seq_indices, table):
    b = query_indices.shape[0]
    v, d = table.shape
    idx = jnp.concatenate(
        [query_indices.astype(jnp.int32), seq_indices.astype(jnp.int32)], axis=1
    )
    total = idx.size
    info = plsc.get_sparse_core_info()
    nc, ns = info.num_cores, info.num_subcores
    nw = nc * ns
    assert total % (nw * CH) == 0
    nchunk = total // (nw * CH)
    idx3 = idx.reshape(nw, nchunk, CH)
    out = _build(nchunk, d, v, nw, nc)(idx3, table)
    return out.reshape(b, -1)
